# initial kernel scaffold (unmeasured)
import jax
import jax.numpy as jnp
from jax import lax
from jax.experimental import pallas as pl
from jax.experimental.pallas import tpu as pltpu

N_DEV = 8
N_LAYERS = 3


def kernel(x, Win0, Wout0, Win1, Wout1, Win2, Wout2):
    b, d = x.shape

    def body(x_ref, win0, wout0, win1, wout1, win2, wout2,
             out_ref, part_ref, gather_ref, send_sems, recv_sems):
        my = lax.axis_index("i")
        wins = [win0, win1, win2]
        wouts = [wout0, wout1, wout2]

        xv = x_ref[...].astype(jnp.bfloat16)
        total = None
        for k in range(N_LAYERS):
            h = jnp.dot(xv, wins[k][...].astype(jnp.bfloat16),
                        preferred_element_type=jnp.float32)
            h = jnp.maximum(h, 0.0).astype(jnp.bfloat16)
            partial = jnp.dot(h, wouts[k][...].astype(jnp.bfloat16),
                              preferred_element_type=jnp.float32)
            part_ref[k] = partial.astype(jnp.bfloat16)

            rdmas = []
            for dist in range(1, N_DEV):
                rdma = pltpu.make_async_remote_copy(
                    src_ref=part_ref.at[k],
                    dst_ref=gather_ref.at[k, dist - 1],
                    send_sem=send_sems.at[k],
                    recv_sem=recv_sems.at[k],
                    device_id=((my + dist) % N_DEV,),
                    device_id_type=pl.DeviceIdType.MESH,
                )
                rdma.start()
                rdmas.append(rdma)
            for rdma in rdmas:
                rdma.wait()

            total = partial
            for dist in range(1, N_DEV):
                total = total + gather_ref[k, dist - 1].astype(jnp.float32)
            xv = total.astype(jnp.bfloat16)

        out_ref[...] = total

    return pl.pallas_call(
        body,
        out_shape=jax.ShapeDtypeStruct((b, d), jnp.float32),
        in_specs=[pl.BlockSpec(memory_space=pltpu.VMEM)] * 7,
        out_specs=pl.BlockSpec(memory_space=pltpu.VMEM),
        scratch_shapes=[
            pltpu.VMEM((N_LAYERS, b, d), jnp.bfloat16),
            pltpu.VMEM((N_LAYERS, N_DEV - 1, b, d), jnp.bfloat16),
            pltpu.SemaphoreType.DMA((N_LAYERS,)),
            pltpu.SemaphoreType.DMA((N_LAYERS,)),
        ],
        compiler_params=pltpu.CompilerParams(collective_id=0),
    )(x, Win0, Wout0, Win1, Wout1, Win2, Wout2)


# baseline (device time: 41468 ns/iter reference)
import jax
import jax.numpy as jnp
from jax import lax
from jax.experimental import pallas as pl
from jax.experimental.pallas import tpu as pltpu

N_DEV = 8
N_LAYERS = 3


def kernel(x, Win0, Wout0, Win1, Wout1, Win2, Wout2):
    b, d = x.shape

    def body(x_ref, win0, wout0, win1, wout1, win2, wout2,
             out_ref, part_ref, gather_ref, send_sems, recv_sems):
        my = lax.axis_index("i")
        wins = [win0, win1, win2]
        wouts = [wout0, wout1, wout2]

        xv = x_ref[...].astype(jnp.bfloat16)
        total = None
        for k in range(N_LAYERS):
            h = jnp.dot(xv, wins[k][...].astype(jnp.bfloat16),
                        preferred_element_type=jnp.float32)
            h = jnp.maximum(h, 0.0).astype(jnp.bfloat16)
            partial = jnp.dot(h, wouts[k][...].astype(jnp.bfloat16),
                              preferred_element_type=jnp.float32)
            part_ref[k] = partial.astype(jnp.bfloat16)

            rdmas = []
            for dist in range(1, N_DEV):
                rdma = pltpu.make_async_remote_copy(
                    src_ref=part_ref.at[k],
                    dst_ref=gather_ref.at[k, dist - 1],
                    send_sem=send_sems.at[k],
                    recv_sem=recv_sems.at[k],
                    device_id=((my + dist) % N_DEV,),
                    device_id_type=pl.DeviceIdType.MESH,
                )
                rdma.start()
                rdmas.append(rdma)
            for rdma in rdmas:
                rdma.wait()

            total = partial
            for dist in range(1, N_DEV):
                total = total + gather_ref[k, dist - 1].astype(jnp.float32)
            xv = total.astype(jnp.bfloat16)

        out_ref[...] = total

    return pl.pallas_call(
        body,
        out_shape=jax.ShapeDtypeStruct((b, d), jnp.float32),
        in_specs=[pl.BlockSpec(memory_space=pltpu.VMEM)] * 7,
        out_specs=pl.BlockSpec(memory_space=pltpu.VMEM),
        scratch_shapes=[
            pltpu.VMEM((N_LAYERS, b, d), jnp.bfloat16),
            pltpu.VMEM((N_LAYERS, N_DEV - 1, b, d), jnp.bfloat16),
            pltpu.SemaphoreType.DMA((N_LAYERS,)),
            pltpu.SemaphoreType.DMA((N_LAYERS,)),
        ],
    )(x, Win0, Wout0, Win1, Wout1, Win2, Wout2)
